# baseline (speedup 1.0000x reference)
import jax
import jax.numpy as jnp
from jax.experimental import pallas as pl
from jax.experimental.pallas import tpu as pltpu

_FINE = 0.7
_MEDIAN = 0.4

_BLK = 4


def _gate_kernel(e_ref, out_ref):
    e = e_ref[...]
    m_gt_med = e > _MEDIAN
    m_gt_fine = e > _FINE
    one = jnp.ones(e.shape, jnp.int32)
    zero = jnp.zeros(e.shape, jnp.int32)
    out_ref[:, 0, :, :] = jnp.where(m_gt_med, zero, one)
    out_ref[:, 1, :, :] = jnp.where(m_gt_med & (~m_gt_fine), one, zero)
    out_ref[:, 2, :, :] = jnp.where(m_gt_fine, one, zero)


_NBUF = 6


def _outer(e_hbm, out_hbm):
    pipe = pltpu.emit_pipeline(
        _gate_kernel,
        grid=(64 // _BLK,),
        in_specs=[pl.BlockSpec((_BLK, 64, 1024), lambda i: (i, 0, 0),
                               pipeline_mode=pl.Buffered(buffer_count=_NBUF))],
        out_specs=[pl.BlockSpec((_BLK, 3, 64, 1024), lambda i: (i, 0, 0, 0),
                                pipeline_mode=pl.Buffered(buffer_count=2))],
    )
    pipe(e_hbm, out_hbm)


def kernel(entropy):
    t = jnp.transpose(entropy, (1, 2, 0))
    out = pl.pallas_call(
        _outer,
        in_specs=[pl.BlockSpec(memory_space=pltpu.MemorySpace.HBM)],
        out_specs=pl.BlockSpec(memory_space=pltpu.MemorySpace.HBM),
        out_shape=jax.ShapeDtypeStruct((64, 3, 64, 1024), jnp.int32),
    )(t)
    return jnp.transpose(out, (3, 0, 2, 1))


# speedup vs baseline: 1.1027x; 1.1027x over previous
import jax
import jax.numpy as jnp
from jax.experimental import pallas as pl
from jax.experimental.pallas import tpu as pltpu

_FINE = 0.7
_MEDIAN = 0.4

_BLK = 8


def _gate_kernel(e_ref, out_ref):
    e = e_ref[...]
    m_gt_med = e > _MEDIAN
    m_gt_fine = e > _FINE
    one = jnp.ones(e.shape, jnp.int32)
    zero = jnp.zeros(e.shape, jnp.int32)
    out_ref[:, 0, :, :] = jnp.where(m_gt_med, zero, one)
    out_ref[:, 1, :, :] = jnp.where(m_gt_med & (~m_gt_fine), one, zero)
    out_ref[:, 2, :, :] = jnp.where(m_gt_fine, one, zero)


_NBUF = 6


def _outer(e_hbm, out_hbm):
    pipe = pltpu.emit_pipeline(
        _gate_kernel,
        grid=(64 // _BLK,),
        in_specs=[pl.BlockSpec((_BLK, 64, 1024), lambda i: (i, 0, 0),
                               pipeline_mode=pl.Buffered(buffer_count=_NBUF))],
        out_specs=[pl.BlockSpec((_BLK, 3, 64, 1024), lambda i: (i, 0, 0, 0),
                                pipeline_mode=pl.Buffered(buffer_count=2))],
    )
    pipe(e_hbm, out_hbm)


def kernel(entropy):
    t = jnp.transpose(entropy, (1, 2, 0))
    out = pl.pallas_call(
        _outer,
        in_specs=[pl.BlockSpec(memory_space=pltpu.MemorySpace.HBM)],
        out_specs=pl.BlockSpec(memory_space=pltpu.MemorySpace.HBM),
        out_shape=jax.ShapeDtypeStruct((64, 3, 64, 1024), jnp.int32),
    )(t)
    return jnp.transpose(out, (3, 0, 2, 1))


# speedup vs baseline: 1.1119x; 1.0083x over previous
import jax
import jax.numpy as jnp
from jax.experimental import pallas as pl
from jax.experimental.pallas import tpu as pltpu

_FINE = 0.7
_MEDIAN = 0.4

_BLK = 8


def _gate_kernel(e_ref, out_ref):
    e = e_ref[...]
    m_gt_med = e > _MEDIAN
    m_gt_fine = e > _FINE
    one = jnp.ones(e.shape, jnp.int32)
    zero = jnp.zeros(e.shape, jnp.int32)
    out_ref[:, 0, :, :] = jnp.where(m_gt_med, zero, one)
    out_ref[:, 1, :, :] = jnp.where(m_gt_med & (~m_gt_fine), one, zero)
    out_ref[:, 2, :, :] = jnp.where(m_gt_fine, one, zero)


_NBUF = 8


def _outer(e_hbm, out_hbm):
    pipe = pltpu.emit_pipeline(
        _gate_kernel,
        grid=(64 // _BLK,),
        in_specs=[pl.BlockSpec((_BLK, 64, 1024), lambda i: (i, 0, 0),
                               pipeline_mode=pl.Buffered(buffer_count=_NBUF))],
        out_specs=[pl.BlockSpec((_BLK, 3, 64, 1024), lambda i: (i, 0, 0, 0),
                                pipeline_mode=pl.Buffered(buffer_count=2))],
    )
    pipe(e_hbm, out_hbm)


def kernel(entropy):
    t = jnp.transpose(entropy, (1, 2, 0))
    out = pl.pallas_call(
        _outer,
        in_specs=[pl.BlockSpec(memory_space=pltpu.MemorySpace.HBM)],
        out_specs=pl.BlockSpec(memory_space=pltpu.MemorySpace.HBM),
        out_shape=jax.ShapeDtypeStruct((64, 3, 64, 1024), jnp.int32),
    )(t)
    return jnp.transpose(out, (3, 0, 2, 1))
